# baseline (speedup 1.0000x reference)
import functools
import math

import jax
import jax.numpy as jnp
from jax import lax
from jax.experimental import pallas as pl
from jax.experimental.pallas import tpu as pltpu
from jax.experimental.pallas import tpu_sc as plsc

DIM = 1024
HEADS = 4
NUM_KEYS = 256
TOPK = 32
DIM_HEAD = 128

B_TOK = 8192
KPT = HEADS * TOPK
NC, NS, NLANE = 2, 16, 16
NW = NC * NS
TPW = B_TOK // NW
GCH = 32
NCH = KPT // GCH


def _sc_bag(values, vi3, wrep):
    mesh = plsc.VectorSubcoreMesh(core_axis_name="c", subcore_axis_name="s")

    @functools.partial(
        pl.kernel,
        out_type=jax.ShapeDtypeStruct((B_TOK, DIM), jnp.float32),
        mesh=mesh,
        scratch_types=[
            pltpu.VMEM((NCH, GCH), jnp.int32),
            pltpu.VMEM((KPT, NLANE), jnp.float32),
            pltpu.VMEM((GCH, DIM), jnp.float32),
            pltpu.VMEM((DIM,), jnp.float32),
            pltpu.SemaphoreType.DMA,
        ],
    )
    def bag(values_hbm, vi_hbm, w_hbm, out_hbm, idx_v, w_v, rows_v, acc_v, sem):
        wid = lax.axis_index("s") * NC + lax.axis_index("c")
        t0 = wid * TPW

        def token_body(i, _):
            t = t0 + i
            pltpu.sync_copy(vi_hbm.at[t], idx_v)
            pltpu.sync_copy(w_hbm.at[t], w_v)
            zero = jnp.zeros((NLANE,), jnp.float32)
            for c in range(DIM // NLANE):
                acc_v[pl.ds(c * NLANE, NLANE)] = zero
            for g in range(NCH):
                pltpu.async_copy(values_hbm.at[idx_v.at[g]], rows_v, sem).wait()

                def row_body(r, _):
                    w = w_v[g * GCH + r]
                    for c in range(DIM // NLANE):
                        chunk = rows_v[r, pl.ds(c * NLANE, NLANE)]
                        plsc.addupdate(acc_v.at[pl.ds(c * NLANE, NLANE)],
                                       chunk * w)
                    return 0

                lax.fori_loop(0, GCH, row_body, 0)
            pltpu.sync_copy(acc_v, out_hbm.at[t])
            return 0

        lax.fori_loop(0, TPW, token_body, 0)

    return bag(values, vi3, wrep)


def _front_end(x, Wq, ln_g, ln_b, keys_p):
    b, t, _ = x.shape
    q = x @ Wq.T
    q = q.reshape(b, t, 2, HEADS, DIM_HEAD)
    mu = q.mean(axis=-1, keepdims=True)
    var = q.var(axis=-1, keepdims=True)
    q = (q - mu) / jnp.sqrt(var + 1e-5) * ln_g + ln_b
    dots = jnp.einsum('btphd,hnpd->bthpn', q, keys_p)
    scores, indices = lax.top_k(dots, TOPK)
    sx, sy = scores[:, :, :, 0], scores[:, :, :, 1]
    ix, iy = indices[:, :, :, 0], indices[:, :, :, 1]
    all_scores = (sx[..., :, None] + sy[..., None, :]).reshape(b, t, HEADS, TOPK * TOPK)
    all_indices = (ix[..., :, None] * NUM_KEYS + iy[..., None, :]).reshape(b, t, HEADS, TOPK * TOPK)
    final_scores, final_pos = lax.top_k(all_scores, TOPK)
    value_indices = jnp.take_along_axis(all_indices, final_pos, axis=-1)
    attn = jax.nn.softmax(final_scores, axis=-1)
    vi = value_indices.reshape(b * t, KPT).astype(jnp.int32)
    at = attn.reshape(b * t, KPT)
    return vi, at


def kernel(x, Wq, ln_g, ln_b, keys_p, values):
    b, t, _ = x.shape
    vi, at = _front_end(x, Wq, ln_g, ln_b, keys_p)
    vi3 = vi.reshape(B_TOK, NCH, GCH)
    wrep = jnp.broadcast_to(at[:, :, None], (B_TOK, KPT, NLANE))
    out = _sc_bag(values, vi3, wrep)
    return out.reshape(b, t, DIM)


# speedup vs baseline: 1.5772x; 1.5772x over previous
import functools

import numpy as np
import jax
import jax.numpy as jnp
from jax import lax
from jax.experimental import pallas as pl
from jax.experimental.pallas import tpu as pltpu
from jax.experimental.pallas import tpu_sc as plsc

DIM = 1024
HEADS = 4
NUM_KEYS = 256
TOPK = 32
DIM_HEAD = 128

B_TOK = 8192
TBLK = 256
IMIN = -2147483648

NC, NS, NLANE = 2, 16, 16
NW = NC * NS
TPW = B_TOK // NW
SKB = 8
GCH = 32
NCH = HEADS


def _tc_body(x_ref, w_ref, k_ref, g_ref, b_ref, out_ref):
    q = jnp.dot(x_ref[...], w_ref[...], preferred_element_type=jnp.float32)
    cols = []
    for c in range(8):
        qc = q[:, c * 128:(c + 1) * 128]
        mu = jnp.mean(qc, axis=1, keepdims=True)
        qd = qc - mu
        var = jnp.mean(qd * qd, axis=1, keepdims=True)
        qn = qd * lax.rsqrt(var + 1e-5)
        qn = qn * g_ref[0:1, :] + b_ref[0:1, :]
        d = jnp.dot(qn, k_ref[c], preferred_element_type=jnp.float32)
        s32 = lax.bitcast_convert_type(d, jnp.int32)
        key = jnp.where(s32 >= 0, s32, IMIN - s32)
        lane = lax.broadcasted_iota(jnp.int32, d.shape, 1)
        key = jnp.bitwise_or(jnp.bitwise_and(key, jnp.int32(-256)), 255 - lane)
        picks = []
        for _ in range(TOPK):
            m = jnp.max(key, axis=1, keepdims=True)
            picks.append(m)
            key = jnp.where(key == m, IMIN, key)
        cols.append(jnp.concatenate(picks, axis=1))
    out_ref[...] = jnp.concatenate(cols, axis=1)


def _tc_front(x2, WqT, KT, g2, b2):
    return pl.pallas_call(
        _tc_body,
        grid=(B_TOK // TBLK,),
        in_specs=[
            pl.BlockSpec((TBLK, DIM), lambda i: (i, 0)),
            pl.BlockSpec((DIM, DIM), lambda i: (0, 0)),
            pl.BlockSpec((8, DIM_HEAD, NUM_KEYS), lambda i: (0, 0, 0)),
            pl.BlockSpec((8, DIM_HEAD), lambda i: (0, 0)),
            pl.BlockSpec((8, DIM_HEAD), lambda i: (0, 0)),
        ],
        out_specs=pl.BlockSpec((TBLK, 256), lambda i: (i, 0)),
        out_shape=jax.ShapeDtypeStruct((B_TOK, 256), jnp.int32),
    )(x2, WqT, KT, g2, b2)


_pairs = [(i, j) for i in range(TOPK) for j in range(TOPK // (i + 1))]
N_CAND = len(_pairs)
_ci_np = np.zeros((8, NLANE), np.int32)
_cj_np = np.zeros((8, NLANE), np.int32)
for _c, (_i, _j) in enumerate(_pairs):
    _ci_np[_c // NLANE, _c % NLANE] = _i
    _cj_np[_c // NLANE, _c % NLANE] = _j


def _unpack(k):
    idx = 255 - jnp.bitwise_and(k, 255)
    cls = jnp.bitwise_and(k, jnp.int32(-256))
    s32 = jnp.where(cls >= 0, cls, IMIN - cls)
    return lax.bitcast_convert_type(s32, jnp.float32), idx


def _sel(m, a, b):
    return jnp.where(m, a, b)


def _merge16(a, b):
    rk = lax.rev(b[0], (0,))
    rv = lax.rev(b[1], (0,))
    m = a[0] >= rk
    hik, hiv = _sel(m, a[0], rk), _sel(m, a[1], rv)
    lok, lov = _sel(m, rk, a[0]), _sel(m, rv, a[1])
    hi = plsc.sort_key_val(hik, hiv, descending=True)
    lo = plsc.sort_key_val(lok, lov, descending=True)
    return hi, lo


def _top32(A, B):
    r0k, r0v = lax.rev(B[1][0], (0,)), lax.rev(B[1][1], (0,))
    r1k, r1v = lax.rev(B[0][0], (0,)), lax.rev(B[0][1], (0,))
    m0 = A[0][0] >= r0k
    n0 = (_sel(m0, A[0][0], r0k), _sel(m0, A[0][1], r0v))
    m1 = A[1][0] >= r1k
    n1 = (_sel(m1, A[1][0], r1k), _sel(m1, A[1][1], r1v))
    m = n0[0] >= n1[0]
    t0 = (_sel(m, n0[0], n1[0]), _sel(m, n0[1], n1[1]))
    t1 = (_sel(m, n1[0], n0[0]), _sel(m, n1[1], n0[1]))
    hi = plsc.sort_key_val(t0[0], t0[1], descending=True)
    lo = plsc.sort_key_val(t1[0], t1[1], descending=True)
    return hi, lo


def _sc_stage2_bag(values, skeys, ci, cj):
    mesh = plsc.VectorSubcoreMesh(core_axis_name="c", subcore_axis_name="s")

    @functools.partial(
        pl.kernel,
        out_type=jax.ShapeDtypeStruct((B_TOK, DIM), jnp.float32),
        mesh=mesh,
        compiler_params=pltpu.CompilerParams(needs_layout_passes=False),
        scratch_types=[
            pltpu.VMEM((SKB, 256), jnp.int32),
            pltpu.VMEM((8, NLANE), jnp.int32),
            pltpu.VMEM((8, NLANE), jnp.int32),
            pltpu.VMEM((TOPK,), jnp.float32),
            pltpu.VMEM((TOPK,), jnp.float32),
            pltpu.VMEM((TOPK,), jnp.int32),
            pltpu.VMEM((TOPK,), jnp.int32),
            pltpu.VMEM((NCH, GCH), jnp.int32),
            pltpu.VMEM((NCH, GCH), jnp.float32),
            pltpu.VMEM((2, GCH, DIM), jnp.float32),
            pltpu.VMEM((DIM,), jnp.float32),
            pltpu.SemaphoreType.DMA,
            pltpu.SemaphoreType.DMA,
        ],
    )
    def body(values_hbm, sk_hbm, ci_hbm, cj_hbm, out_hbm,
             sk_v, ci_v, cj_v, sx_v, sy_v, ix_v, iy_v,
             idx2_v, w2_v, rows_v, acc_v, sem0, sem1):
        sems = (sem0, sem1)
        wid = lax.axis_index("s") * NC + lax.axis_index("c")
        t0 = wid * TPW
        pltpu.sync_copy(ci_hbm, ci_v)
        pltpu.sync_copy(cj_hbm, cj_v)

        def stage2(tt):
            for h in range(HEADS):
                kx0, ix0 = _unpack(sk_v[tt, pl.ds(h * 32, 16)])
                kx1, ix1 = _unpack(sk_v[tt, pl.ds(h * 32 + 16, 16)])
                ky0, iy0 = _unpack(sk_v[tt, pl.ds((4 + h) * 32, 16)])
                ky1, iy1 = _unpack(sk_v[tt, pl.ds((4 + h) * 32 + 16, 16)])
                sx_v[pl.ds(0, 16)] = kx0
                sx_v[pl.ds(16, 16)] = kx1
                sy_v[pl.ds(0, 16)] = ky0
                sy_v[pl.ds(16, 16)] = ky1
                ix_v[pl.ds(0, 16)] = ix0
                ix_v[pl.ds(16, 16)] = ix1
                iy_v[pl.ds(0, 16)] = iy0
                iy_v[pl.ds(16, 16)] = iy1
                groups = []
                lanes = lax.iota(jnp.int32, 16)
                for gc in range(8):
                    cig = ci_v[gc]
                    cjg = cj_v[gc]
                    cs = (plsc.load_gather(sx_v, [cig])
                          + plsc.load_gather(sy_v, [cjg]))
                    if gc == 7:
                        cs = jnp.where(lanes < N_CAND - 112, cs, -3e38)
                    vix = plsc.load_gather(ix_v, [cig])
                    viy = plsc.load_gather(iy_v, [cjg])
                    vidx = vix * 256 + viy
                    groups.append(plsc.sort_key_val(cs, vidx, descending=True))
                p0 = _merge16(groups[0], groups[1])
                p1 = _merge16(groups[2], groups[3])
                p2 = _merge16(groups[4], groups[5])
                p3 = _merge16(groups[6], groups[7])
                q0 = _top32(p0, p1)
                q1 = _top32(p2, p3)
                fhi, flo = _top32(q0, q1)
                mx = jnp.max(fhi[0])
                e0 = jnp.exp(fhi[0] - mx)
                e1 = jnp.exp(flo[0] - mx)
                denom = jnp.full((16,), jnp.sum(e0) + jnp.sum(e1))
                w2_v[h, pl.ds(0, 16)] = e0 / denom
                w2_v[h, pl.ds(16, 16)] = e1 / denom
                idx2_v[h, pl.ds(0, 16)] = fhi[1]
                idx2_v[h, pl.ds(16, 16)] = flo[1]

        def accumulate(buf, g):
            def row_body(r, _):
                w = plsc.load_gather(
                    w2_v, [jnp.full((16,), g, jnp.int32),
                           jnp.full((16,), r, jnp.int32)])
                for c in range(DIM // NLANE):
                    chunk = rows_v[buf, r, pl.ds(c * NLANE, NLANE)]
                    plsc.addupdate(acc_v.at[pl.ds(c * NLANE, NLANE)],
                                   chunk * w)
                return 0
            lax.fori_loop(0, GCH, row_body, 0)

        def token_body(i, _):
            tt = i % SKB

            @pl.when(tt == 0)
            def _():
                pltpu.sync_copy(
                    sk_hbm.at[pl.ds(pl.multiple_of(t0 + i, SKB), SKB)], sk_v)

            stage2(tt)
            zero = jnp.zeros((NLANE,), jnp.float32)
            for c in range(DIM // NLANE):
                acc_v[pl.ds(c * NLANE, NLANE)] = zero
            cp0 = pltpu.async_copy(values_hbm.at[idx2_v.at[0]],
                                   rows_v.at[0], sems[0])
            cp1 = pltpu.async_copy(values_hbm.at[idx2_v.at[1]],
                                   rows_v.at[1], sems[1])
            cp0.wait()
            accumulate(0, 0)
            cp2 = pltpu.async_copy(values_hbm.at[idx2_v.at[2]],
                                   rows_v.at[0], sems[0])
            cp1.wait()
            accumulate(1, 1)
            cp3 = pltpu.async_copy(values_hbm.at[idx2_v.at[3]],
                                   rows_v.at[1], sems[1])
            cp2.wait()
            accumulate(0, 2)
            cp3.wait()
            accumulate(1, 3)
            pltpu.sync_copy(acc_v, out_hbm.at[t0 + i])
            return 0

        lax.fori_loop(0, TPW, token_body, 0)

    return body(values, skeys, ci, cj)


def kernel(x, Wq, ln_g, ln_b, keys_p, values):
    b, t, _ = x.shape
    x2 = x.reshape(b * t, DIM)
    WqT = Wq.T
    KT = jnp.transpose(keys_p, (2, 0, 3, 1)).reshape(8, DIM_HEAD, NUM_KEYS)
    g2 = jnp.broadcast_to(ln_g[None, :], (8, DIM_HEAD))
    b2 = jnp.broadcast_to(ln_b[None, :], (8, DIM_HEAD))
    skeys = _tc_front(x2, WqT, KT, g2, b2)
    ci = jnp.asarray(_ci_np)
    cj = jnp.asarray(_cj_np)
    out = _sc_stage2_bag(values, skeys, ci, cj)
    return out.reshape(b, t, DIM)
